# baseline (speedup 1.0000x reference)
import functools

import jax
import jax.numpy as jnp
from jax import lax
from jax.experimental import pallas as pl
from jax.experimental.pallas import tpu as pltpu
from jax.experimental.pallas import tpu_sc as plsc

N_DATA = 50000
N_TRUNC = 10000
E = 150000
D = 512

NS = 16
NC = 2
NW = NC * NS
G = 128
GPW = 37
E_W = GPW * G
E_PAD = NW * E_W
GP2 = 2 * GPW

BSHIFT = 12
BROWS = 1 << BSHIFT
CAPG = E_PAD // G + NW
NBKT_DN = 3
NBKT_UP = 13

N_TRUNC_ROWS = 80
N_DATA_ROWS = 400

_mesh = functools.partial(
    plsc.VectorSubcoreMesh, core_axis_name="c", subcore_axis_name="s")
_params = pltpu.CompilerParams(needs_layout_passes=False)


def _zvec():
    return jnp.zeros((16,), jnp.float32)


def _make_norm_kernel():
    @functools.partial(
        pl.kernel,
        mesh=_mesh(),
        compiler_params=_params,
        out_type=(
            jax.ShapeDtypeStruct((N_TRUNC_ROWS, 128), jnp.float32),
            jax.ShapeDtypeStruct((N_DATA_ROWS, 128), jnp.float32),
        ),
        scratch_types=[
            pltpu.VMEM((GP2, G), jnp.int32),
            pltpu.VMEM((GP2, G), jnp.float32),
            pltpu.VMEM((N_DATA_ROWS, 128), jnp.float32),
            pltpu.VMEM((1, N_TRUNC_ROWS), jnp.int32),
            pltpu.VMEM((1, N_DATA_ROWS), jnp.int32),
            pltpu.VMEM_SHARED((N_DATA_ROWS, 128), jnp.float32),
        ],
    )
    def norm_kernel(dstd_h, wd_h, dstu_h, wu_h, out_d, out_u,
                    dst_v, w_v, part_v, iota_d_v, iota_u_v, acc_sh):
        c = lax.axis_index("c")
        s = lax.axis_index("s")

        def iota_d_body(i, carry):
            iota_d_v[0, pl.ds(i * 16, 16)] = lax.iota(jnp.int32, 16) + i * 16
            return carry
        lax.fori_loop(0, N_TRUNC_ROWS // 16, iota_d_body, 0)

        def iota_u_body(i, carry):
            iota_u_v[0, pl.ds(i * 16, 16)] = lax.iota(jnp.int32, 16) + i * 16
            return carry
        lax.fori_loop(0, N_DATA_ROWS // 16, iota_u_body, 0)

        def run(dst_h, w_h, out_h, n_rows, iota_v):
            pltpu.sync_copy(dst_h.at[s], dst_v)
            pltpu.sync_copy(w_h.at[s], w_v)

            def zero_body(i, carry):
                r = i // 8
                j = (i % 8) * 16
                part_v[r, pl.ds(j, 16)] = _zvec()
                return carry
            lax.fori_loop(0, n_rows * 8, zero_body, 0)

            def acc_body(i, carry):
                g = i // 8
                j = (i % 8) * 16
                dv = dst_v[g, pl.ds(j, 16)]
                wv = w_v[g, pl.ds(j, 16)]
                row = lax.shift_right_logical(dv, 7)
                col = lax.bitwise_and(dv, 127)
                plsc.addupdate_scatter(part_v, [row, col], wv)
                return carry
            lax.fori_loop(0, GP2 * 8, acc_body, 0)

            @pl.when(s == 0)
            def _():
                pltpu.sync_copy(part_v.at[pl.ds(0, n_rows)],
                                acc_sh.at[pl.ds(0, n_rows)])
            plsc.subcore_barrier()

            @pl.when(s != 0)
            def _():
                pltpu.sync_copy(part_v.at[pl.ds(0, n_rows)],
                                acc_sh.at[iota_v.at[0]], add=True)
            plsc.subcore_barrier()

            nch = n_rows // 8
            for k in range((nch + NS - 1) // NS):
                @pl.when(s + k * NS < nch)
                def _():
                    off = pl.multiple_of((s + k * NS) * 8, 8)
                    pltpu.sync_copy(acc_sh.at[pl.ds(off, 8)],
                                    out_h.at[pl.ds(off, 8)])

        @pl.when(c == 0)
        def _():
            run(dstd_h, wd_h, out_d, N_TRUNC_ROWS, iota_d_v)

        @pl.when(c == 1)
        def _():
            run(dstu_h, wu_h, out_u, N_DATA_ROWS, iota_u_v)

    return norm_kernel


def _make_count_kernel(nbkt, n_norm):
    @functools.partial(
        pl.kernel,
        mesh=_mesh(),
        compiler_params=_params,
        out_type=(
            jax.ShapeDtypeStruct((NW * 32,), jnp.int32),
            jax.ShapeDtypeStruct((NW, GPW, G), jnp.float32),
        ),
        scratch_types=[
            pltpu.VMEM((GPW, G), jnp.int32),
            pltpu.VMEM((GPW, G), jnp.float32),
            pltpu.VMEM((32,), jnp.int32),
            pltpu.VMEM((n_norm,), jnp.float32),
        ],
    )
    def count_kernel(dst_h, w_h, norm_h, cnt_h, wn_h,
                     dst_v, w_v, cnt_v, norm_v):
        c = lax.axis_index("c")
        s = lax.axis_index("s")
        w = s * NC + c

        pltpu.sync_copy(dst_h.at[w], dst_v)
        pltpu.sync_copy(w_h.at[w], w_v)
        pltpu.sync_copy(norm_h, norm_v)

        cnt_v[pl.ds(0, 16)] = jnp.zeros((16,), jnp.int32)
        cnt_v[pl.ds(16, 16)] = jnp.zeros((16,), jnp.int32)
        ones = jnp.ones((16,), jnp.int32)

        def body(i, carry):
            g = i // 8
            j = (i % 8) * 16
            dv = dst_v[g, pl.ds(j, 16)]
            bv = lax.shift_right_logical(dv, BSHIFT)
            plsc.addupdate_scatter(cnt_v, [bv], ones)
            nv = plsc.load_gather(norm_v, [dv])
            w_v[g, pl.ds(j, 16)] = w_v[g, pl.ds(j, 16)] / (nv + 1e-8)
            return carry
        lax.fori_loop(0, GPW * 8, body, 0)

        pltpu.sync_copy(cnt_v, cnt_h.at[pl.ds(w * 32, 32)])
        pltpu.sync_copy(w_v, wn_h.at[w])

    return count_kernel


def _make_bin_kernel(nbkt):
    rcap = GPW + 1

    @functools.partial(
        pl.kernel,
        mesh=_mesh(),
        compiler_params=_params,
        out_type=jax.ShapeDtypeStruct((nbkt * CAPG * 384,), jnp.int32),
        scratch_types=[
            pltpu.VMEM((GPW, G), jnp.int32),
            pltpu.VMEM((GPW, G), jnp.int32),
            pltpu.VMEM((GPW, G), jnp.float32),
            pltpu.VMEM((1, 128), jnp.int32),
            pltpu.VMEM((rcap * 384,), jnp.int32),
        ],
    )
    def bin_kernel(src_h, dst_h, w_h, offg_h, out_h,
                   src_v, dst_v, w_v, offg_v, run_v):
        c = lax.axis_index("c")
        s = lax.axis_index("s")
        w = s * NC + c

        pltpu.sync_copy(src_h.at[w], src_v)
        pltpu.sync_copy(dst_h.at[w], dst_v)
        pltpu.sync_copy(w_h.at[w], w_v)
        pltpu.sync_copy(offg_h.at[w], offg_v)

        off_lo = offg_v[0, pl.ds(0, 16)]
        off_hi = offg_v[0, pl.ds(16, 16)]
        zero16 = jnp.zeros((16,), jnp.int32)
        iota16 = lax.iota(jnp.int32, 16)

        for b in range(nbkt):
            off_b = off_lo[b] if b < 16 else off_hi[b - 16]

            def scan_body(i, cnt):
                g = i // 8
                j = (i % 8) * 16
                dv = dst_v[g, pl.ds(j, 16)]
                m = lax.shift_right_logical(dv, BSHIFT) == b
                pc = plsc.all_reduce_population_count(m)[0]

                @pl.when(pc > 0)
                def _():
                    mi = jnp.where(m, 1, 0)
                    cs = plsc.cumsum(mi)
                    pos = cnt + cs - 1
                    base = lax.shift_right_logical(pos, 7) * 384
                    lane = lax.bitwise_and(pos, 127)
                    sv = src_v[g, pl.ds(j, 16)]
                    wv = plsc.bitcast(w_v[g, pl.ds(j, 16)], jnp.int32)
                    plsc.store_scatter(run_v, [base + lane], sv, mask=m)
                    plsc.store_scatter(run_v, [base + 128 + lane], dv, mask=m)
                    plsc.store_scatter(run_v, [base + 256 + lane], wv, mask=m)
                return cnt + pc
            cnt = lax.fori_loop(0, GPW * 8, scan_body, 0)

            ng = (cnt + 127) // 128
            for j in range(8):
                posj = cnt + j * 16 + iota16
                mt = posj < ng * 128
                basej = lax.shift_right_logical(posj, 7) * 384
                lane = lax.bitwise_and(posj, 127)
                plsc.store_scatter(run_v, [basej + 256 + lane], zero16,
                                   mask=mt)

            def dma_body(q, carry):
                pltpu.sync_copy(
                    run_v.at[pl.ds(q * 384, 384)],
                    out_h.at[pl.ds((off_b + q) * 384, 384)])
                return carry
            lax.fori_loop(0, ng, dma_body, 0)

    return bin_kernel


def _make_spmm_kernel(nbkt, n_src):
    nb_core = (nbkt + NC - 1) // NC
    n_out_rows = nbkt * BROWS

    zbpt = BROWS // NS // G

    @functools.partial(
        pl.kernel,
        mesh=_mesh(),
        compiler_params=_params,
        out_type=jax.ShapeDtypeStruct((4, n_out_rows, 128), jnp.float32),
        scratch_types=[
            pltpu.VMEM((2 * 384,), jnp.int32),
            pltpu.VMEM((2, 128), jnp.int32),
            pltpu.VMEM((2, 128), jnp.int32),
            pltpu.VMEM((2, G, 128), jnp.float32),
            pltpu.VMEM((G, 128), jnp.float32),
            pltpu.VMEM((32,), jnp.int32),
            pltpu.VMEM_SHARED((BROWS, 128), jnp.float32),
            pltpu.SemaphoreType.DMA,
        ],
    )
    def spmm_kernel(table_h, rec_h, gcnt_h, out_h,
                    eblk_v, gidx_v, dl_v, rows_v, zb_v, gcnt_v,
                    acc_sh, gsem):
        c = lax.axis_index("c")
        s = lax.axis_index("s")

        pltpu.sync_copy(gcnt_h, gcnt_v)

        def zb_body(i, carry):
            r = i // 8
            j = (i % 8) * 16
            zb_v[r, pl.ds(j, 16)] = _zvec()
            return carry
        lax.fori_loop(0, G * 8, zb_body, 0)

        def bucket_body(p, bcarry):
            b = c * nb_core + p

            @pl.when(b < nbkt)
            def _():
                ngv = plsc.load_gather(gcnt_v, [jnp.full((16,), b, jnp.int32)])
                ng = ngv[0]
                bbase = b * CAPG
                trips = (ng - s + NS - 1) // NS

                def load_idx(t, buf):
                    ebase = buf * 384
                    pltpu.sync_copy(
                        rec_h.at[pl.ds((bbase + s + t * NS) * 384, 384)],
                        eblk_v.at[pl.ds(ebase, 384)])

                    def idx_body(i, carry2):
                        j = i * 16
                        sv = eblk_v[pl.ds(ebase + j, 16)]
                        dv = eblk_v[pl.ds(ebase + 128 + j, 16)]
                        gidx_v[buf, pl.ds(j, 16)] = (
                            jnp.clip(sv, 0, n_src - 1) + k * n_src)
                        dl_v[buf, pl.ds(j, 16)] = lax.bitwise_and(
                            dv, BROWS - 1)
                        return carry2
                    lax.fori_loop(0, 8, idx_body, 0)

                for k in range(4):
                    for q in range(zbpt):
                        pltpu.sync_copy(
                            zb_v, acc_sh.at[pl.ds(
                                pl.multiple_of((s * zbpt + q) * G, 8), G)])
                    plsc.subcore_barrier()

                    @pl.when(trips > 0)
                    def _():
                        load_idx(0, 0)
                        pltpu.async_copy(table_h.at[gidx_v.at[0]],
                                        rows_v.at[0], gsem)

                    def group_body(t, carry):
                        cur = lax.bitwise_and(t, 1)
                        nxt = 1 - cur
                        ebase = cur * 384

                        pltpu.make_async_copy(
                            table_h.at[pl.ds(0, G)], rows_v.at[cur],
                            gsem).wait()

                        @pl.when(t + 1 < trips)
                        def _():
                            load_idx(t + 1, nxt)
                            pltpu.async_copy(table_h.at[gidx_v.at[nxt]],
                                            rows_v.at[nxt], gsem)

                        def wvec_body(jw, carry2):
                            wb = plsc.bitcast(
                                eblk_v[pl.ds(ebase + 256 + jw * 16, 16)],
                                jnp.float32)
                            for l in range(16):
                                wsp = jnp.full((16,), wb[l], jnp.float32)
                                r = jw * 16 + l
                                for f in range(8):
                                    sl = pl.ds(f * 16, 16)
                                    rows_v[cur, r, sl] = (
                                        rows_v[cur, r, sl] * wsp)
                            return carry2
                        lax.fori_loop(0, 8, wvec_body, 0)

                        pltpu.sync_copy(rows_v.at[cur],
                                        acc_sh.at[dl_v.at[cur]], add=True)
                        return carry
                    lax.fori_loop(0, trips, group_body, 0)
                    plsc.subcore_barrier()

                    for q in range(zbpt):
                        roff = pl.multiple_of((s * zbpt + q) * G, 8)
                        pltpu.sync_copy(
                            acc_sh.at[pl.ds(roff, G)],
                            out_h.at[k, pl.ds(
                                pl.multiple_of(b * BROWS, 8) + roff, G)])
                    plsc.subcore_barrier()
            return bcarry

        lax.fori_loop(0, nb_core, bucket_body, 0)

    return spmm_kernel


_norm = _make_norm_kernel()
_count_dn = _make_count_kernel(NBKT_DN, N_TRUNC_ROWS * 128)
_count_up = _make_count_kernel(NBKT_UP, N_DATA_ROWS * 128)
_bin_dn = _make_bin_kernel(NBKT_DN)
_bin_up = _make_bin_kernel(NBKT_UP)
_spmm_dn = _make_spmm_kernel(NBKT_DN, N_DATA)
_spmm_up = _make_spmm_kernel(NBKT_UP, N_TRUNC)


def _prep_edges(a, shape):
    a = jnp.concatenate([a, jnp.zeros((E_PAD - E,), a.dtype)])
    return a.reshape(*shape)


def _bin_offsets(cnt, nbkt):
    cnt = cnt.reshape(NW, 32)
    gpad = (cnt + G - 1) // G
    start = jnp.cumsum(gpad, axis=0) - gpad
    base = jnp.arange(32, dtype=jnp.int32) * CAPG
    offg = (base[None, :] + start).astype(jnp.int32)
    gcnt = jnp.sum(gpad, axis=0).astype(jnp.int32)
    offg = jnp.pad(offg, ((0, 0), (0, 96)))
    return offg.reshape(NW, 1, 128), gcnt


def _project(x_flat, src, dst, w, norm_flat, count_k, bin_k, spmm_k,
             nbkt, n_src):
    src32 = src.reshape(NW, GPW, G)
    dst32 = dst.reshape(NW, GPW, G)
    w32 = w.reshape(NW, GPW, G)
    cnt, wn = count_k(dst32, w32, norm_flat)
    offg, gcnt = _bin_offsets(cnt, nbkt)
    recs = bin_k(src32, dst32, wn, offg)
    return spmm_k(x_flat, recs, gcnt)


def kernel(x, src_down, dst_down, src_up, dst_up, w_down, w_up):
    xg = x[0, -1, 0]
    xflat = xg.reshape(N_DATA, 4, 128).transpose(1, 0, 2).reshape(
        4 * N_DATA, 128)

    sd = _prep_edges(src_down, (NS, GP2, G))
    dd = _prep_edges(dst_down, (NS, GP2, G))
    wd = _prep_edges(w_down, (NS, GP2, G))
    su = _prep_edges(src_up, (NS, GP2, G))
    du = _prep_edges(dst_up, (NS, GP2, G))
    wu = _prep_edges(w_up, (NS, GP2, G))

    norm_d, norm_u = _norm(dd, wd, du, wu)
    norm_d = norm_d.reshape(N_TRUNC_ROWS * 128)
    norm_u = norm_u.reshape(N_DATA_ROWS * 128)

    coarse = _project(xflat, sd, dd, wd, norm_d,
                      _count_dn, _bin_dn, _spmm_dn, NBKT_DN, N_DATA)
    coarse_flat = coarse[:, :N_TRUNC, :].reshape(4 * N_TRUNC, 128)

    fine4 = _project(coarse_flat, su, du, wu, norm_u,
                     _count_up, _bin_up, _spmm_up, NBKT_UP, N_TRUNC)
    fine = fine4[:, :N_DATA, :].transpose(1, 0, 2).reshape(N_DATA, D)
    return fine.reshape(1, 1, N_DATA, D)


# speedup vs baseline: 1.3416x; 1.3416x over previous
import functools

import jax
import jax.numpy as jnp
from jax import lax
from jax.experimental import pallas as pl
from jax.experimental.pallas import tpu as pltpu
from jax.experimental.pallas import tpu_sc as plsc

N_DATA = 50000
N_TRUNC = 10000
E = 150000
D = 512

NS = 16
NC = 2
NW = NC * NS
G = 128
GPW = 37
E_W = GPW * G
E_PAD = NW * E_W
GP2 = 2 * GPW

BSHIFT = 12
BROWS = 1 << BSHIFT
CAPG = E_PAD // G + NW
NBKT_DN = 3
NBKT_UP = 13

N_TRUNC_ROWS = 80
N_DATA_ROWS = 400

_mesh = functools.partial(
    plsc.VectorSubcoreMesh, core_axis_name="c", subcore_axis_name="s")
_params = pltpu.CompilerParams(needs_layout_passes=False)


def _zvec():
    return jnp.zeros((16,), jnp.float32)


def _make_norm_kernel():
    @functools.partial(
        pl.kernel,
        mesh=_mesh(),
        compiler_params=_params,
        out_type=(
            jax.ShapeDtypeStruct((N_TRUNC_ROWS, 128), jnp.float32),
            jax.ShapeDtypeStruct((N_DATA_ROWS, 128), jnp.float32),
        ),
        scratch_types=[
            pltpu.VMEM((GP2, G), jnp.int32),
            pltpu.VMEM((GP2, G), jnp.float32),
            pltpu.VMEM((N_DATA_ROWS, 128), jnp.float32),
            pltpu.VMEM((1, N_TRUNC_ROWS), jnp.int32),
            pltpu.VMEM((1, N_DATA_ROWS), jnp.int32),
            pltpu.VMEM_SHARED((N_DATA_ROWS, 128), jnp.float32),
        ],
    )
    def norm_kernel(dstd_h, wd_h, dstu_h, wu_h, out_d, out_u,
                    dst_v, w_v, part_v, iota_d_v, iota_u_v, acc_sh):
        c = lax.axis_index("c")
        s = lax.axis_index("s")

        def iota_d_body(i, carry):
            iota_d_v[0, pl.ds(i * 16, 16)] = lax.iota(jnp.int32, 16) + i * 16
            return carry
        lax.fori_loop(0, N_TRUNC_ROWS // 16, iota_d_body, 0)

        def iota_u_body(i, carry):
            iota_u_v[0, pl.ds(i * 16, 16)] = lax.iota(jnp.int32, 16) + i * 16
            return carry
        lax.fori_loop(0, N_DATA_ROWS // 16, iota_u_body, 0)

        def run(dst_h, w_h, out_h, n_rows, iota_v):
            pltpu.sync_copy(dst_h.at[s], dst_v)
            pltpu.sync_copy(w_h.at[s], w_v)

            def zero_body(i, carry):
                r = i // 8
                j = (i % 8) * 16
                part_v[r, pl.ds(j, 16)] = _zvec()
                return carry
            lax.fori_loop(0, n_rows * 8, zero_body, 0)

            def acc_body(i, carry):
                g = i // 8
                j = (i % 8) * 16
                dv = dst_v[g, pl.ds(j, 16)]
                wv = w_v[g, pl.ds(j, 16)]
                row = lax.shift_right_logical(dv, 7)
                col = lax.bitwise_and(dv, 127)
                plsc.addupdate_scatter(part_v, [row, col], wv)
                return carry
            lax.fori_loop(0, GP2 * 8, acc_body, 0)

            @pl.when(s == 0)
            def _():
                pltpu.sync_copy(part_v.at[pl.ds(0, n_rows)],
                                acc_sh.at[pl.ds(0, n_rows)])
            plsc.subcore_barrier()

            @pl.when(s != 0)
            def _():
                pltpu.sync_copy(part_v.at[pl.ds(0, n_rows)],
                                acc_sh.at[iota_v.at[0]], add=True)
            plsc.subcore_barrier()

            nch = n_rows // 8
            for k in range((nch + NS - 1) // NS):
                @pl.when(s + k * NS < nch)
                def _():
                    off = pl.multiple_of((s + k * NS) * 8, 8)
                    pltpu.sync_copy(acc_sh.at[pl.ds(off, 8)],
                                    out_h.at[pl.ds(off, 8)])

        @pl.when(c == 0)
        def _():
            run(dstd_h, wd_h, out_d, N_TRUNC_ROWS, iota_d_v)

        @pl.when(c == 1)
        def _():
            run(dstu_h, wu_h, out_u, N_DATA_ROWS, iota_u_v)

    return norm_kernel


def _make_count_kernel(nbkt, n_norm):
    @functools.partial(
        pl.kernel,
        mesh=_mesh(),
        compiler_params=_params,
        out_type=(
            jax.ShapeDtypeStruct((NW * 32,), jnp.int32),
            jax.ShapeDtypeStruct((NW, GPW, G), jnp.float32),
        ),
        scratch_types=[
            pltpu.VMEM((GPW, G), jnp.int32),
            pltpu.VMEM((GPW, G), jnp.float32),
            pltpu.VMEM((32,), jnp.int32),
            pltpu.VMEM((n_norm,), jnp.float32),
        ],
    )
    def count_kernel(dst_h, w_h, norm_h, cnt_h, wn_h,
                     dst_v, w_v, cnt_v, norm_v):
        c = lax.axis_index("c")
        s = lax.axis_index("s")
        w = s * NC + c

        pltpu.sync_copy(dst_h.at[w], dst_v)
        pltpu.sync_copy(w_h.at[w], w_v)
        pltpu.sync_copy(norm_h, norm_v)

        cnt_v[pl.ds(0, 16)] = jnp.zeros((16,), jnp.int32)
        cnt_v[pl.ds(16, 16)] = jnp.zeros((16,), jnp.int32)
        ones = jnp.ones((16,), jnp.int32)

        def body(i, carry):
            g = i // 8
            j = (i % 8) * 16
            dv = dst_v[g, pl.ds(j, 16)]
            bv = lax.shift_right_logical(dv, BSHIFT)
            plsc.addupdate_scatter(cnt_v, [bv], ones)
            nv = plsc.load_gather(norm_v, [dv])
            w_v[g, pl.ds(j, 16)] = w_v[g, pl.ds(j, 16)] / (nv + 1e-8)
            return carry
        lax.fori_loop(0, GPW * 8, body, 0)

        pltpu.sync_copy(cnt_v, cnt_h.at[pl.ds(w * 32, 32)])
        pltpu.sync_copy(w_v, wn_h.at[w])

    return count_kernel


def _make_bin_kernel(nbkt):
    rcap = GPW + 1

    @functools.partial(
        pl.kernel,
        mesh=_mesh(),
        compiler_params=_params,
        out_type=jax.ShapeDtypeStruct((nbkt * CAPG * 384,), jnp.int32),
        scratch_types=[
            pltpu.VMEM((GPW, G), jnp.int32),
            pltpu.VMEM((GPW, G), jnp.int32),
            pltpu.VMEM((GPW, G), jnp.float32),
            pltpu.VMEM((1, 128), jnp.int32),
            pltpu.VMEM((rcap * 384,), jnp.int32),
        ],
    )
    def bin_kernel(src_h, dst_h, w_h, offg_h, out_h,
                   src_v, dst_v, w_v, offg_v, run_v):
        c = lax.axis_index("c")
        s = lax.axis_index("s")
        w = s * NC + c

        pltpu.sync_copy(src_h.at[w], src_v)
        pltpu.sync_copy(dst_h.at[w], dst_v)
        pltpu.sync_copy(w_h.at[w], w_v)
        pltpu.sync_copy(offg_h.at[w], offg_v)

        off_lo = offg_v[0, pl.ds(0, 16)]
        off_hi = offg_v[0, pl.ds(16, 16)]
        zero16 = jnp.zeros((16,), jnp.int32)
        iota16 = lax.iota(jnp.int32, 16)

        for b in range(nbkt):
            off_b = off_lo[b] if b < 16 else off_hi[b - 16]

            def scan_body(i, cnt):
                g = i // 8
                j = (i % 8) * 16
                dv = dst_v[g, pl.ds(j, 16)]
                m = lax.shift_right_logical(dv, BSHIFT) == b
                pc = plsc.all_reduce_population_count(m)[0]

                @pl.when(pc > 0)
                def _():
                    mi = jnp.where(m, 1, 0)
                    cs = plsc.cumsum(mi)
                    pos = cnt + cs - 1
                    base = lax.shift_right_logical(pos, 7) * 384
                    lane = lax.bitwise_and(pos, 127)
                    sv = src_v[g, pl.ds(j, 16)]
                    wv = plsc.bitcast(w_v[g, pl.ds(j, 16)], jnp.int32)
                    plsc.store_scatter(run_v, [base + lane], sv, mask=m)
                    plsc.store_scatter(run_v, [base + 128 + lane], dv, mask=m)
                    plsc.store_scatter(run_v, [base + 256 + lane], wv, mask=m)
                return cnt + pc
            cnt = lax.fori_loop(0, GPW * 8, scan_body, 0)

            ng = (cnt + 127) // 128
            for j in range(8):
                posj = cnt + j * 16 + iota16
                mt = posj < ng * 128
                basej = lax.shift_right_logical(posj, 7) * 384
                lane = lax.bitwise_and(posj, 127)
                plsc.store_scatter(run_v, [basej + 256 + lane], zero16,
                                   mask=mt)

            def dma_body(q, carry):
                pltpu.sync_copy(
                    run_v.at[pl.ds(q * 384, 384)],
                    out_h.at[pl.ds((off_b + q) * 384, 384)])
                return carry
            lax.fori_loop(0, ng, dma_body, 0)

    return bin_kernel


def _make_spmm_kernel(nbkt, n_src):
    nb_core = (nbkt + NC - 1) // NC
    n_out_rows = nbkt * BROWS

    zbpt = BROWS // NS // G

    @functools.partial(
        pl.kernel,
        mesh=_mesh(),
        compiler_params=_params,
        out_type=jax.ShapeDtypeStruct((4, n_out_rows, 128), jnp.float32),
        scratch_types=[
            pltpu.VMEM((2 * 384,), jnp.int32),
            pltpu.VMEM((1, 128), jnp.int32),
            pltpu.VMEM((1, 128), jnp.int32),
            pltpu.VMEM((1, 128), jnp.int32),
            pltpu.VMEM((1, 128), jnp.int32),
            pltpu.VMEM((G, 128), jnp.float32),
            pltpu.VMEM((G, 128), jnp.float32),
            pltpu.VMEM((G, 128), jnp.float32),
            pltpu.VMEM((32,), jnp.int32),
            pltpu.VMEM_SHARED((BROWS, 128), jnp.float32),
            pltpu.SemaphoreType.DMA,
        ],
    )
    def spmm_kernel(table_h, rec_h, gcnt_h, out_h,
                    eblk_v, gidx0_v, gidx1_v, dl0_v, dl1_v,
                    rows0_v, rows1_v, zb_v, gcnt_v, acc_sh, gsem):
        c = lax.axis_index("c")
        s = lax.axis_index("s")

        pltpu.sync_copy(gcnt_h, gcnt_v)

        def zb_body(i, carry):
            r = i // 8
            j = (i % 8) * 16
            zb_v[r, pl.ds(j, 16)] = _zvec()
            return carry
        lax.fori_loop(0, G * 8, zb_body, 0)

        def bucket_body(p, bcarry):
            b = c * nb_core + p

            @pl.when(b < nbkt)
            def _():
                ngv = plsc.load_gather(gcnt_v, [jnp.full((16,), b, jnp.int32)])
                ng = ngv[0]
                bbase = b * CAPG
                trips = (ng - s + NS - 1) // NS

                bufs = ((0, gidx0_v, dl0_v, rows0_v),
                        (384, gidx1_v, dl1_v, rows1_v))

                def load_idx(t, ebase, gidx_v, dl_v):
                    pltpu.sync_copy(
                        rec_h.at[pl.ds((bbase + s + t * NS) * 384, 384)],
                        eblk_v.at[pl.ds(ebase, 384)])

                    def idx_body(i, carry2):
                        j = i * 16
                        sv = eblk_v[pl.ds(ebase + j, 16)]
                        dv = eblk_v[pl.ds(ebase + 128 + j, 16)]
                        gidx_v[0, pl.ds(j, 16)] = (
                            jnp.clip(sv, 0, n_src - 1) + k * n_src)
                        dl_v[0, pl.ds(j, 16)] = lax.bitwise_and(
                            dv, BROWS - 1)
                        return carry2
                    lax.fori_loop(0, 8, idx_body, 0)

                for k in range(4):
                    for q in range(zbpt):
                        pltpu.sync_copy(
                            zb_v, acc_sh.at[pl.ds(
                                pl.multiple_of((s * zbpt + q) * G, 8), G)])
                    plsc.subcore_barrier()

                    @pl.when(trips > 0)
                    def _():
                        load_idx(0, 0, gidx0_v, dl0_v)
                        pltpu.async_copy(table_h.at[gidx0_v.at[0]],
                                        rows0_v, gsem)

                    def pair_body(qq, carry):
                        for par in range(2):
                            t = 2 * qq + par
                            ebase, _, dl_v, rows_v = bufs[par]
                            _, ngidx_v, ndl_v, nrows_v = bufs[1 - par]
                            nebase = bufs[1 - par][0]

                            @pl.when(t < trips)
                            def _():
                                pltpu.make_async_copy(
                                    table_h.at[pl.ds(0, G)], rows_v,
                                    gsem).wait()

                                @pl.when(t + 1 < trips)
                                def _():
                                    load_idx(t + 1, nebase, ngidx_v, ndl_v)
                                    pltpu.async_copy(
                                        table_h.at[ngidx_v.at[0]],
                                        nrows_v, gsem)

                                def wvec_body(jw, carry2):
                                    wb = plsc.bitcast(
                                        eblk_v[pl.ds(ebase + 256 + jw * 16,
                                                     16)],
                                        jnp.float32)
                                    for l in range(16):
                                        wsp = jnp.full((16,), wb[l],
                                                       jnp.float32)
                                        r = jw * 16 + l
                                        for f in range(8):
                                            sl = pl.ds(f * 16, 16)
                                            rows_v[r, sl] = rows_v[r, sl] * wsp
                                    return carry2
                                lax.fori_loop(0, 8, wvec_body, 0)

                                pltpu.sync_copy(rows_v,
                                                acc_sh.at[dl_v.at[0]],
                                                add=True)
                        return carry
                    lax.fori_loop(0, (trips + 1) // 2, pair_body, 0)
                    plsc.subcore_barrier()

                    for q in range(zbpt):
                        roff = pl.multiple_of((s * zbpt + q) * G, 8)
                        pltpu.sync_copy(
                            acc_sh.at[pl.ds(roff, G)],
                            out_h.at[k, pl.ds(
                                pl.multiple_of(b * BROWS, 8) + roff, G)])
                    plsc.subcore_barrier()
            return bcarry

        lax.fori_loop(0, nb_core, bucket_body, 0)

    return spmm_kernel


_norm = _make_norm_kernel()
_count_dn = _make_count_kernel(NBKT_DN, N_TRUNC_ROWS * 128)
_count_up = _make_count_kernel(NBKT_UP, N_DATA_ROWS * 128)
_bin_dn = _make_bin_kernel(NBKT_DN)
_bin_up = _make_bin_kernel(NBKT_UP)
_spmm_dn = _make_spmm_kernel(NBKT_DN, N_DATA)
_spmm_up = _make_spmm_kernel(NBKT_UP, N_TRUNC)


def _prep_edges(a, shape):
    a = jnp.concatenate([a, jnp.zeros((E_PAD - E,), a.dtype)])
    return a.reshape(*shape)


def _bin_offsets(cnt, nbkt):
    cnt = cnt.reshape(NW, 32)
    gpad = (cnt + G - 1) // G
    start = jnp.cumsum(gpad, axis=0) - gpad
    base = jnp.arange(32, dtype=jnp.int32) * CAPG
    offg = (base[None, :] + start).astype(jnp.int32)
    gcnt = jnp.sum(gpad, axis=0).astype(jnp.int32)
    offg = jnp.pad(offg, ((0, 0), (0, 96)))
    return offg.reshape(NW, 1, 128), gcnt


def _project(x_flat, src, dst, w, norm_flat, count_k, bin_k, spmm_k,
             nbkt, n_src):
    src32 = src.reshape(NW, GPW, G)
    dst32 = dst.reshape(NW, GPW, G)
    w32 = w.reshape(NW, GPW, G)
    cnt, wn = count_k(dst32, w32, norm_flat)
    offg, gcnt = _bin_offsets(cnt, nbkt)
    recs = bin_k(src32, dst32, wn, offg)
    return spmm_k(x_flat, recs, gcnt)


def kernel(x, src_down, dst_down, src_up, dst_up, w_down, w_up):
    xg = x[0, -1, 0]
    xflat = xg.reshape(N_DATA, 4, 128).transpose(1, 0, 2).reshape(
        4 * N_DATA, 128)

    sd = _prep_edges(src_down, (NS, GP2, G))
    dd = _prep_edges(dst_down, (NS, GP2, G))
    wd = _prep_edges(w_down, (NS, GP2, G))
    su = _prep_edges(src_up, (NS, GP2, G))
    du = _prep_edges(dst_up, (NS, GP2, G))
    wu = _prep_edges(w_up, (NS, GP2, G))

    norm_d, norm_u = _norm(dd, wd, du, wu)
    norm_d = norm_d.reshape(N_TRUNC_ROWS * 128)
    norm_u = norm_u.reshape(N_DATA_ROWS * 128)

    coarse = _project(xflat, sd, dd, wd, norm_d,
                      _count_dn, _bin_dn, _spmm_dn, NBKT_DN, N_DATA)
    coarse_flat = coarse[:, :N_TRUNC, :].reshape(4 * N_TRUNC, 128)

    fine4 = _project(coarse_flat, su, du, wu, norm_u,
                     _count_up, _bin_up, _spmm_up, NBKT_UP, N_TRUNC)
    fine = fine4[:, :N_DATA, :].transpose(1, 0, 2).reshape(N_DATA, D)
    return fine.reshape(1, 1, N_DATA, D)
